# baseline (speedup 1.0000x reference)
import functools

import jax
import jax.numpy as jnp
from jax import lax
from jax.experimental import pallas as pl
from jax.experimental.pallas import tpu as pltpu
from jax.experimental.pallas import tpu_sc as plsc

DIM = 768
INTER = 256
E = 8
SI = 512
T = 2048
P = 2 * T
STEPS = 40
BLK = 128
NB = T // BLK
MT = P // BLK
NC, NS = 2, 16
NW = NC * NS
TPW = T // NW

_HI = jax.lax.Precision.HIGHEST


def _route_kernel(x_ref, gw_ref, p0_ref, p1_ref, w0_ref, w1_ref, meta_ref):
    xf = x_ref[...]
    scores = jax.nn.sigmoid(
        lax.dot_general(xf, gw_ref[...], (((1,), (1,)), ((), ())),
                        preferred_element_type=jnp.float32))
    lane8 = lax.broadcasted_iota(jnp.int32, (T, E), 1)
    m1 = jnp.max(scores, axis=1, keepdims=True)
    i1 = jnp.argmax(scores, axis=1)[:, None]
    masked = jnp.where(lane8 == i1, -jnp.inf, scores)
    m2 = jnp.max(masked, axis=1, keepdims=True)
    i2 = jnp.argmax(masked, axis=1)[:, None]
    denom = m1 + m2
    lane16 = jnp.ones((1, 16), jnp.float32)
    w0_ref[...] = (m1 / denom) * lane16
    w1_ref[...] = (m2 / denom) * lane16

    oh0 = (lane8 == i1).astype(jnp.float32)
    oh1 = (lane8 == i2).astype(jnp.float32)

    c16 = jnp.concatenate([oh0, oh1], axis=1)
    ri = lax.broadcasted_iota(jnp.int32, (BLK, BLK), 0)
    ci = lax.broadcasted_iota(jnp.int32, (BLK, BLK), 1)
    ltri = (ri >= ci).astype(jnp.float32)
    parts = []
    running = jnp.zeros((1, 2 * E), jnp.float32)
    for b in range(NB):
        blk = c16[b * BLK:(b + 1) * BLK, :]
        incl = lax.dot(ltri, blk, precision=_HI,
                       preferred_element_type=jnp.float32)
        parts.append(incl - blk + running)
        running = running + incl[BLK - 1:BLK, :]
    rex = jnp.concatenate(parts, axis=0)
    r0, r1 = rex[:, :E], rex[:, E:]

    cnt0 = jnp.sum(oh0, axis=0, keepdims=True)
    cnt = cnt0 + jnp.sum(oh1, axis=0, keepdims=True)
    e_r = lax.broadcasted_iota(jnp.int32, (E, E), 0)
    e_c = lax.broadcasted_iota(jnp.int32, (E, E), 1)
    utri = (e_r <= e_c).astype(jnp.float32)
    off_incl = lax.dot(cnt, utri, precision=_HI,
                       preferred_element_type=jnp.float32)
    off_excl = off_incl - cnt

    p0_ref[...] = jnp.sum(oh0 * (off_excl + r0), axis=1,
                          keepdims=True).astype(jnp.int32)
    p1_ref[...] = jnp.sum(oh1 * (off_excl + cnt0 + r1), axis=1,
                          keepdims=True).astype(jnp.int32)

    eye8 = (e_r == e_c).astype(jnp.float32)
    offx_col = jnp.sum(eye8 * off_excl, axis=1, keepdims=True)
    offi_col = jnp.sum(eye8 * off_incl, axis=1, keepdims=True)
    mlane = lax.broadcasted_iota(jnp.int32, (1, MT), 1).astype(jnp.float32) * BLK
    ovT = ((offx_col < mlane + BLK) & (offi_col > mlane)
           ).astype(jnp.float32)
    nT = jnp.sum(ovT, axis=0, keepdims=True)
    m_r = lax.broadcasted_iota(jnp.int32, (MT, MT), 0)
    m_c = lax.broadcasted_iota(jnp.int32, (MT, MT), 1)
    start_m = lax.dot(nT, (m_r < m_c).astype(jnp.float32), precision=_HI,
                      preferred_element_type=jnp.float32)
    total = jnp.sum(nT)
    eyeN = (m_r == m_c).astype(jnp.float32)
    start_col = jnp.sum(eyeN * start_m, axis=1, keepdims=True)

    svec = lax.broadcasted_iota(jnp.int32, (1, STEPS), 1).astype(jnp.float32)
    m_s = jnp.sum((start_col <= svec).astype(jnp.float32), axis=0,
                  keepdims=True) - 1.0
    sub32 = lax.broadcasted_iota(jnp.int32, (MT, STEPS), 0).astype(jnp.float32)
    m1hot = (sub32 == m_s).astype(jnp.float32)
    start_s = jnp.sum(m1hot * start_col, axis=0, keepdims=True)
    r_s = svec - start_s
    ov_s = lax.dot(ovT, m1hot, precision=_HI,
                   preferred_element_type=jnp.float32)
    ltri8 = (e_r > e_c).astype(jnp.float32)
    rank_s = lax.dot(ltri8, ov_s, precision=_HI,
                     preferred_element_type=jnp.float32)
    pick = ((ov_s > 0.5) & (rank_s == r_s)).astype(jnp.float32)
    sub8 = lax.broadcasted_iota(jnp.int32, (E, STEPS), 0).astype(jnp.float32)
    e_s = jnp.sum(pick * sub8, axis=0, keepdims=True)
    off_s = jnp.sum(pick * offx_col, axis=0, keepdims=True)
    end_s = jnp.sum(pick * offi_col, axis=0, keepdims=True)
    valid = svec < total
    e_last = jnp.sum(jnp.where(svec == total - 1.0, e_s, 0.0))
    e_s = jnp.where(valid, e_s, e_last)
    lo_s = jnp.maximum(off_s - m_s * BLK, 0.0)
    hi_s = jnp.clip(end_s - m_s * BLK, 0.0, float(BLK))
    lo_s = jnp.where(valid, lo_s, 0.0)
    hi_s = jnp.where(valid, hi_s, 0.0)
    zpad = jnp.zeros((E - 4, STEPS), jnp.float32)
    meta_ref[...] = jnp.concatenate(
        [m_s, e_s, lo_s, hi_s, zpad], axis=0).astype(jnp.int32)


def _shared_kernel(x_ref, sw13_ref, sw2_ref, z_ref):
    xb = x_ref[...].astype(jnp.bfloat16)
    ab = lax.dot(xb, sw13_ref[...], preferred_element_type=jnp.float32)
    hs = (jax.nn.silu(ab[:, :SI]) * ab[:, SI:]).astype(jnp.bfloat16)
    z_ref[...] = lax.dot(hs, sw2_ref[...], preferred_element_type=jnp.float32)


def _gmm_kernel(meta_ref, xs_ref, w13_ref, w2_ref, ys_ref):
    s = pl.program_id(0)
    lo = meta_ref[2, s]
    hi = meta_ref[3, s]

    @pl.when(hi > 0)
    def _do():
        riota = lax.broadcasted_iota(jnp.int32, (BLK, 1), 0)
        maskf = ((riota >= lo) & (riota < hi)).astype(jnp.float32)
        xb = (xs_ref[...] * maskf).astype(jnp.bfloat16)
        ab = lax.dot(xb, w13_ref[0], preferred_element_type=jnp.float32)
        h = (jax.nn.silu(ab[:, :INTER]) * ab[:, INTER:]).astype(jnp.bfloat16)
        y = lax.dot(h, w2_ref[0], preferred_element_type=jnp.float32)

        @pl.when(lo == 0)
        def _init():
            ys_ref[...] = y

        @pl.when(lo > 0)
        def _acc():
            ys_ref[...] += y


_HTOK = TPW // 2


@functools.cache
def _sc_kernels():
    mesh = plsc.VectorSubcoreMesh(core_axis_name="c", subcore_axis_name="s",
                                  num_cores=NC, num_subcores=NS)

    @functools.partial(
        pl.kernel,
        out_type=jax.ShapeDtypeStruct((P, DIM), jnp.float32),
        mesh=mesh,
        scratch_types=[
            pltpu.VMEM((TPW, DIM), jnp.float32),
            pltpu.VMEM((TPW,), jnp.int32),
            pltpu.VMEM((TPW,), jnp.int32),
        ],
    )
    def dispatch_kernel(x_hbm, p0_hbm, p1_hbm, xs_hbm, x_v, i0_v, i1_v):
        wid = lax.axis_index("s") * NC + lax.axis_index("c")
        base = pl.multiple_of(wid * TPW, TPW)
        pltpu.sync_copy(x_hbm.at[pl.ds(base, TPW)], x_v)
        pltpu.sync_copy(p0_hbm.at[pl.ds(base, TPW)], i0_v)
        pltpu.sync_copy(p1_hbm.at[pl.ds(base, TPW)], i1_v)
        pltpu.sync_copy(x_v, xs_hbm.at[i0_v])
        pltpu.sync_copy(x_v, xs_hbm.at[i1_v])

    @functools.partial(
        pl.kernel,
        out_type=jax.ShapeDtypeStruct((T, DIM), jnp.float32),
        mesh=mesh,
        scratch_types=[
            pltpu.VMEM((_HTOK, DIM), jnp.float32),
            pltpu.VMEM((_HTOK, DIM), jnp.float32),
            pltpu.VMEM((_HTOK, DIM), jnp.float32),
            pltpu.VMEM((_HTOK, DIM), jnp.float32),
            pltpu.VMEM((_HTOK,), jnp.int32),
            pltpu.VMEM((_HTOK,), jnp.int32),
            pltpu.VMEM((_HTOK, 16), jnp.float32),
            pltpu.VMEM((_HTOK, 16), jnp.float32),
        ],
    )
    def combine_kernel(ys_hbm, z_hbm, p0_hbm, p1_hbm, w0_hbm, w1_hbm, out_hbm,
                       r0_v, r1_v, z_v, o_v, i0_v, i1_v, w0_v, w1_v):
        wid = lax.axis_index("s") * NC + lax.axis_index("c")
        for c in range(2):
            base = pl.multiple_of(wid * TPW + c * _HTOK, _HTOK)
            pltpu.sync_copy(p0_hbm.at[pl.ds(base, _HTOK)], i0_v)
            pltpu.sync_copy(p1_hbm.at[pl.ds(base, _HTOK)], i1_v)
            pltpu.sync_copy(w0_hbm.at[pl.ds(base, _HTOK)], w0_v)
            pltpu.sync_copy(w1_hbm.at[pl.ds(base, _HTOK)], w1_v)
            pltpu.sync_copy(z_hbm.at[pl.ds(base, _HTOK)], z_v)
            pltpu.sync_copy(ys_hbm.at[i0_v], r0_v)
            pltpu.sync_copy(ys_hbm.at[i1_v], r1_v)

            def tok(i, _):
                a = w0_v[i, :]
                b = w1_v[i, :]
                for j in range(DIM // 16):
                    sl = pl.ds(j * 16, 16)
                    o_v[i, sl] = (a * r0_v[i, sl] + b * r1_v[i, sl]
                                  + z_v[i, sl])
                return 0

            lax.fori_loop(0, _HTOK, tok, 0)
            pltpu.sync_copy(o_v, out_hbm.at[pl.ds(base, _HTOK)])

    return dispatch_kernel, combine_kernel


@jax.jit
def kernel(x, gate_w, w1, w2, w3, sw1, sw2, sw3):
    shape = x.shape
    xt = x.reshape(-1, DIM)
    w13 = jnp.concatenate([w1, w3], axis=2).astype(jnp.bfloat16)
    w2b = w2.astype(jnp.bfloat16)
    sw13 = jnp.concatenate([sw1, sw3], axis=1).astype(jnp.bfloat16)
    sw2b = sw2.astype(jnp.bfloat16)

    full = lambda shp: pl.BlockSpec(shp, lambda *_: (0,) * len(shp))

    p0, p1, w0, w1r, meta = pl.pallas_call(
        _route_kernel,
        in_specs=[full((T, DIM)), full((E, DIM))],
        out_specs=[full((T, 1)), full((T, 1)), full((T, 16)), full((T, 16)),
                   full((E, STEPS))],
        out_shape=[
            jax.ShapeDtypeStruct((T, 1), jnp.int32),
            jax.ShapeDtypeStruct((T, 1), jnp.int32),
            jax.ShapeDtypeStruct((T, 16), jnp.float32),
            jax.ShapeDtypeStruct((T, 16), jnp.float32),
            jax.ShapeDtypeStruct((E, STEPS), jnp.int32),
        ],
    )(xt, gate_w)
    p0f = p0.reshape(T)
    p1f = p1.reshape(T)
    w0f = w0
    w1f = w1r

    z = pl.pallas_call(
        _shared_kernel,
        in_specs=[full((T, DIM)), full((DIM, 2 * SI)), full((SI, DIM))],
        out_specs=full((T, DIM)),
        out_shape=jax.ShapeDtypeStruct((T, DIM), jnp.float32),
    )(xt, sw13, sw2b)

    dispatch_kernel, combine_kernel = _sc_kernels()
    xs = dispatch_kernel(xt, p0f, p1f)

    ys = pl.pallas_call(
        _gmm_kernel,
        grid_spec=pltpu.PrefetchScalarGridSpec(
            num_scalar_prefetch=1,
            grid=(STEPS,),
            in_specs=[
                pl.BlockSpec((BLK, DIM), lambda s, m: (m[0, s], 0)),
                pl.BlockSpec((1, DIM, 2 * INTER), lambda s, m: (m[1, s], 0, 0)),
                pl.BlockSpec((1, INTER, DIM), lambda s, m: (m[1, s], 0, 0)),
            ],
            out_specs=pl.BlockSpec((BLK, DIM), lambda s, m: (m[0, s], 0)),
        ),
        out_shape=jax.ShapeDtypeStruct((P, DIM), jnp.float32),
    )(meta, xs, w13, w2b)

    out = combine_kernel(ys, z, p0f, p1f, w0f, w1f)
    return out.reshape(shape)


# speedup vs baseline: 2.5495x; 2.5495x over previous
import jax
import jax.numpy as jnp
from jax.experimental import pallas as pl
from jax.experimental.pallas import tpu as pltpu

DIM = 768
INTER = 256
E = 8
SI = 512
T = 2048
TS = T // E


def _moe_kernel(x_ref, gw_ref, w13_ref, w2_ref, sw13_ref, sw2_ref,
                out_ref, combine_ref, xb_ref):
    e = pl.program_id(0)

    @pl.when(e == 0)
    def _init():
        xf = x_ref[...]
        xb_ref[...] = xf.astype(jnp.bfloat16)
        scores = jax.nn.sigmoid(
            jax.lax.dot_general(xf, gw_ref[...], (((1,), (1,)), ((), ())),
                                preferred_element_type=jnp.float32))
        m1 = jnp.max(scores, axis=1, keepdims=True)
        i1 = jnp.argmax(scores, axis=1)[:, None]
        eids = jax.lax.broadcasted_iota(jnp.int32, (T, E), 1)
        masked = jnp.where(eids == i1, -jnp.inf, scores)
        m2 = jnp.max(masked, axis=1, keepdims=True)
        i2 = jnp.argmax(masked, axis=1)[:, None]
        denom = m1 + m2
        combine_ref[...] = (jnp.where(eids == i1, m1 / denom, 0.0)
                            + jnp.where(eids == i2, m2 / denom, 0.0))
        xb = xb_ref[...]
        ab = jax.lax.dot(xb, sw13_ref[...], preferred_element_type=jnp.float32)
        hs = (jax.nn.silu(ab[:, :SI]) * ab[:, SI:]).astype(jnp.bfloat16)
        out_ref[...] = jax.lax.dot(hs, sw2_ref[...],
                                   preferred_element_type=jnp.float32)

    xb = xb_ref[...]
    cmb = combine_ref[...]
    lane = jax.lax.broadcasted_iota(jnp.int32, (T, E), 1)
    ce = jnp.sum(jnp.where(lane == e, cmb, 0.0), axis=1, keepdims=True)
    ab = jax.lax.dot(xb, w13_ref[0], preferred_element_type=jnp.float32)
    h = (jax.nn.silu(ab[:, :INTER]) * ab[:, INTER:] * ce).astype(jnp.bfloat16)
    out_ref[...] += jax.lax.dot(h, w2_ref[0], preferred_element_type=jnp.float32)


@jax.jit
def kernel(x, gate_w, w1, w2, w3, sw1, sw2, sw3):
    shape = x.shape
    xt = x.reshape(-1, DIM)
    w13 = jnp.concatenate([w1, w3], axis=2).astype(jnp.bfloat16)
    w2b = w2.astype(jnp.bfloat16)
    sw13 = jnp.concatenate([sw1, sw3], axis=1).astype(jnp.bfloat16)
    sw2b = sw2.astype(jnp.bfloat16)

    full = lambda shp: pl.BlockSpec(shp, lambda e: (0,) * len(shp))
    per_e = lambda shp: pl.BlockSpec((1,) + shp, lambda e: (e, 0, 0))

    out = pl.pallas_call(
        _moe_kernel,
        grid=(E,),
        in_specs=[
            full((T, DIM)),
            full((E, DIM)),
            per_e((DIM, 2 * INTER)),
            per_e((INTER, DIM)),
            full((DIM, 2 * SI)),
            full((SI, DIM)),
        ],
        out_specs=full((T, DIM)),
        out_shape=jax.ShapeDtypeStruct((T, DIM), jnp.float32),
        scratch_shapes=[
            pltpu.VMEM((T, E), jnp.float32),
            pltpu.VMEM((T, DIM), jnp.bfloat16),
        ],
    )(xt, gate_w, w13, w2b, sw13, sw2b)
    return out.reshape(shape)
